# baseline (speedup 1.0000x reference)
import functools

import jax
import jax.numpy as jnp
from jax import lax
from jax.experimental import pallas as pl
from jax.experimental.pallas import tpu as pltpu
from jax.experimental.pallas import tpu_sc as plsc

NUM_WORKERS = 32
CHUNK = 1600


def _sc_gather_add(table, idx, pos):
    n, d = idx.shape[0], table.shape[1]
    maxlen = pos.shape[0]
    reps = CHUNK // maxlen
    per_w = n // NUM_WORKERS
    n_chunks = per_w // CHUNK
    mesh = plsc.VectorSubcoreMesh(core_axis_name="c", subcore_axis_name="s")

    @functools.partial(
        pl.kernel,
        mesh=mesh,
        out_type=jax.ShapeDtypeStruct((n, d), jnp.float32),
        compiler_params=pltpu.CompilerParams(use_tc_tiling_on_sc=False),
        scratch_types=[
            pltpu.VMEM((CHUNK,), jnp.int32),
            pltpu.VMEM((CHUNK,), jnp.int32),
            pltpu.VMEM((CHUNK, d), jnp.float32),
            pltpu.VMEM((CHUNK, d), jnp.float32),
            pltpu.VMEM((maxlen, d), jnp.float32),
            pltpu.SemaphoreType.DMA,
            pltpu.SemaphoreType.DMA,
            pltpu.SemaphoreType.DMA,
            pltpu.SemaphoreType.DMA,
        ],
    )
    def gather_kernel(table_hbm, idx_hbm, pos_hbm, out_hbm,
                      idx0, idx1, rows0, rows1, pos_v, g0, g1, w0, w1):
        wid = lax.axis_index("s") * 2 + lax.axis_index("c")
        base = wid * per_w
        idx_v = (idx0, idx1)
        rows_v = (rows0, rows1)
        gsem = (g0, g1)
        wsem = (w0, w1)

        pltpu.sync_copy(pos_hbm, pos_v)

        def add_pos(b):
            rows = rows_v[b]

            @pl.loop(0, maxlen)
            def _(p):
                for h in range(d // 16):
                    pv = pos_v[p, pl.ds(h * 16, 16)]
                    for r in range(reps):
                        plsc.addupdate(
                            rows.at[r * maxlen + p, pl.ds(h * 16, 16)], pv)

        def start_gather(ci, b):
            off = base + ci * CHUNK
            pltpu.sync_copy(idx_hbm.at[pl.ds(off, CHUNK)], idx_v[b])
            pltpu.async_copy(table_hbm.at[idx_v[b]], rows_v[b], gsem[b])

        def wait_gather(b):
            pltpu.make_async_copy(table_hbm.at[idx_v[b]], rows_v[b],
                                  gsem[b]).wait()

        def start_writeback(ci, b):
            off = base + ci * CHUNK
            pltpu.async_copy(rows_v[b], out_hbm.at[pl.ds(off, CHUNK)], wsem[b])

        def wait_writeback(ci, b):
            off = base + ci * CHUNK
            pltpu.make_async_copy(rows_v[b], out_hbm.at[pl.ds(off, CHUNK)],
                                  wsem[b]).wait()

        start_gather(0, 0)

        @pl.loop(0, n_chunks, step=2)
        def _(ci):
            for b in range(2):
                cur = ci + b
                nxt = cur + 1

                @pl.when(nxt < n_chunks)
                def _():
                    @pl.when(nxt >= 2)
                    def _():
                        wait_writeback(nxt - 2, 1 - b)
                    start_gather(nxt, 1 - b)

                wait_gather(b)
                add_pos(b)
                start_writeback(cur, b)

        wait_writeback(n_chunks - 2, 0)
        wait_writeback(n_chunks - 1, 1)

    return gather_kernel(table, idx, pos)


def kernel(x, token_table, pos_table):
    b, maxlen = x.shape
    d = token_table.shape[1]
    xf = x.reshape(-1).astype(jnp.int32)
    out = _sc_gather_add(token_table, xf, pos_table)
    return out.reshape(b, maxlen, d)
